# baseline (speedup 1.0000x reference)
import functools

import jax
import jax.numpy as jnp
from jax import lax
from jax.experimental import pallas as pl
from jax.experimental.pallas import tpu as pltpu
from jax.experimental.pallas import tpu_sc as plsc

NC = 2
NS = 16
NW = NC * NS
L = 16
F = 16
WCHUNK = 64
NRING = 2
WBLOCKS = 782


def _mf_kernel_body(task_hbm, worker_hbm, tfT_hbm, wfT_hbm, out_hbm,
                    wpack_hbm, tidx_v, widx_v, wblk_v, tring_v, wfetch_v,
                    wpk_v, wbuf_v, trows_v, wrows_v, out_v, sems):
    b_per_w = tidx_v.shape[0]
    n_groups = b_per_w // L
    n_wchunks = b_per_w // WCHUNK
    sid = lax.axis_index("s")
    cid = lax.axis_index("c")
    wid = sid * NC + cid
    wpack_c = wpack_hbm.at[cid]
    base = wid * b_per_w

    pltpu.sync_copy(task_hbm.at[pl.ds(base, b_per_w)], tidx_v)
    pltpu.sync_copy(worker_hbm.at[pl.ds(base, b_per_w)], widx_v)

    row_iota = lax.iota(jnp.int32, L)

    def t_fire(g, ring):
        tv = tidx_v[pl.ds(g * L, L)]
        for j in range(L):
            blk = pl.multiple_of(
                lax.shift_right_logical(tv[j], 7) * 128, 128)
            pltpu.async_copy(tfT_hbm.at[:, pl.ds(blk, 128)],
                             tring_v.at[ring, j], sems.at[ring])

    def t_drain(ring):
        for j in range(L):
            pltpu.make_async_copy(tfT_hbm.at[:, pl.ds(0, 128)],
                                  tring_v.at[ring, j], sems.at[ring]).wait()

    def w_fetch(t, slot):
        b = sid + 16 * t
        @pl.when(b < WBLOCKS)
        def _():
            c0 = pl.multiple_of(b * 128, 128)
            pltpu.async_copy(wfT_hbm.at[:, pl.ds(c0, 128)],
                             wfetch_v.at[slot], sems.at[NRING + slot])

    def w_transpose(t, slot):
        b = sid + 16 * t
        @pl.when(b < WBLOCKS)
        def _():
            pltpu.make_async_copy(wfT_hbm.at[:, pl.ds(0, 128)],
                                  wfetch_v.at[slot],
                                  sems.at[NRING + slot]).wait()
            for r in range(L):
                for kk in range(8):
                    cidx = jnp.full((L,), 8 * r + kk, jnp.int32)
                    col = plsc.load_gather(wfetch_v.at[slot],
                                           [row_iota, cidx])
                    wpk_v[slot, r, pl.ds(kk * F, F)] = col
            pltpu.sync_copy(wpk_v.at[slot], wpack_c.at[pl.ds(b * L, L)])

    @pl.loop(0, b_per_w, step=L)
    def _(g):
        wblk_v[pl.ds(g, L)] = lax.shift_right_logical(widx_v[pl.ds(g, L)], 3)

    def w_start(c, buf):
        sl = pl.ds(c * WCHUNK, WCHUNK)
        return pltpu.async_copy(wpack_c.at[wblk_v.at[sl]], wbuf_v.at[buf],
                                sems.at[NRING + 2 + buf])

    def w_extract(c, buf):
        @pl.loop(0, WCHUNK, step=L)
        def _(g):
            wv = widx_v[pl.ds(c * WCHUNK + g, L)]
            for j in range(L):
                sub = lax.bitwise_and(wv[j], 7)
                cidx = row_iota + sub * F
                ridx = jnp.full((L,), g + j, jnp.int32)
                wrow = plsc.load_gather(wbuf_v.at[buf], [ridx, cidx])
                wrows_v[pl.ds((c * WCHUNK + g + j) * F, F)] = wrow

    for r in range(NRING):
        t_fire(r, r)
    w_fetch(0, 0)
    w_fetch(1, 1)

    @pl.loop(0, 50, step=2)
    def _(t):
        for dt in range(2):
            w_transpose(t + dt, dt)
            w_fetch(t + dt + 2, dt)

    def t_group(g, ring, with_blocks):
        del with_blocks
        t_drain(ring)
        tv = tidx_v[pl.ds(g * L, L)]
        for j in range(L):
            col = lax.bitwise_and(tv[j], 127)
            cidx = jnp.full((L,), col, jnp.int32)
            tcol = plsc.load_gather(tring_v.at[ring, j], [row_iota, cidx])
            trows_v[pl.ds((g * L + j) * F, F)] = tcol

        @pl.when(g + NRING < n_groups)
        def _():
            t_fire(g + NRING, ring)

    @pl.loop(0, n_groups, step=NRING)
    def _(g):
        for r in range(NRING):
            t_group(g + r, r, False)

    plsc.subcore_barrier()

    wcp = w_start(0, 0)
    for c in range(n_wchunks):
        nxt = w_start(c + 1, 1 - c % 2) if c + 1 < n_wchunks else None
        wcp.wait()
        w_extract(c, c % 2)
        wcp = nxt

    lane16 = row_iota * F

    @pl.loop(0, b_per_w, step=L)
    def _(p0):
        acc = jnp.zeros((L,), jnp.float32)
        for f in range(F):
            idx = lane16 + (p0 * F + f)
            tcol = plsc.load_gather(trows_v, [idx])
            wcol = plsc.load_gather(wrows_v, [idx])
            acc = acc + tcol * wcol
        out_v[pl.ds(p0, L)] = 1.0 / (1.0 + jnp.exp(-acc))

    pltpu.sync_copy(out_v, out_hbm.at[pl.ds(base, b_per_w)])


@jax.jit
def _mf_forward(task, worker, task_factors, worker_factors):
    B = task.shape[0]
    b_per_w = B // NW
    tfT = task_factors.T
    wfT = worker_factors.T
    mesh = plsc.VectorSubcoreMesh(core_axis_name="c", subcore_axis_name="s")
    cp = pltpu.CompilerParams(needs_layout_passes=False,
                              use_tc_tiling_on_sc=True)
    kern = functools.partial(
        pl.kernel,
        compiler_params=cp,
        out_type=(jax.ShapeDtypeStruct((B,), jnp.float32),
                  jax.ShapeDtypeStruct((NC, 12512, 8 * F), jnp.float32)),
        mesh=mesh,
        scratch_types=[
            pltpu.VMEM((b_per_w,), jnp.int32),
            pltpu.VMEM((b_per_w,), jnp.int32),
            pltpu.VMEM((b_per_w,), jnp.int32),
            pltpu.VMEM((NRING, L, F, 128), jnp.float32),
            pltpu.VMEM((2, F, 128), jnp.float32),
            pltpu.VMEM((2, L, 8 * F), jnp.float32),
            pltpu.VMEM((2, WCHUNK, 8 * F), jnp.float32),
            pltpu.VMEM((b_per_w * F,), jnp.float32),
            pltpu.VMEM((b_per_w * F,), jnp.float32),
            pltpu.VMEM((b_per_w,), jnp.float32),
            pltpu.SemaphoreType.DMA((NRING + 4,)),
        ],
    )(_mf_kernel_body)
    return kern(task, worker, tfT, wfT)[0]


def kernel(task, worker, task_factors, worker_factors):
    return _mf_forward(task, worker, task_factors, worker_factors)


# speedup vs baseline: 1.4245x; 1.4245x over previous
import functools

import jax
import jax.numpy as jnp
from jax import lax
from jax.experimental import pallas as pl
from jax.experimental.pallas import tpu as pltpu
from jax.experimental.pallas import tpu_sc as plsc

NC = 2
NS = 16
NW = NC * NS
L = 16
F = 16
WCHUNK = 64
NRING = 3


def _mf_kernel_body(task_hbm, worker_hbm, tfT_hbm, wfp_hbm, out_hbm,
                    tidx_v, widx_v, wblk_v, tring_v, wbuf_v,
                    tg_v, wrows_v, out_v, sems):
    b_per_w = tidx_v.shape[0]
    n_groups = b_per_w // L
    n_wchunks = b_per_w // WCHUNK
    wid = lax.axis_index("s") * NC + lax.axis_index("c")
    base = wid * b_per_w

    pltpu.sync_copy(task_hbm.at[pl.ds(base, b_per_w)], tidx_v)
    pltpu.sync_copy(worker_hbm.at[pl.ds(base, b_per_w)], widx_v)

    row_iota = lax.iota(jnp.int32, L)
    lane16 = row_iota * F

    def t_fire(g, ring):
        tv = tidx_v[pl.ds(g * L, L)]
        for j in range(L):
            blk = pl.multiple_of(
                lax.shift_right_logical(tv[j], 7) * 128, 128)
            pltpu.async_copy(tfT_hbm.at[:, pl.ds(blk, 128)],
                             tring_v.at[ring, j], sems.at[ring])

    def t_drain(ring):
        for j in range(L):
            pltpu.make_async_copy(tfT_hbm.at[:, pl.ds(0, 128)],
                                  tring_v.at[ring, j], sems.at[ring]).wait()

    @pl.loop(0, b_per_w, step=L)
    def _(g):
        wblk_v[pl.ds(g, L)] = lax.shift_right_logical(widx_v[pl.ds(g, L)], 3)

    def w_start(c, buf):
        sl = pl.ds(c * WCHUNK, WCHUNK)
        return pltpu.async_copy(wfp_hbm.at[wblk_v.at[sl]], wbuf_v.at[buf],
                                sems.at[NRING + buf])

    def w_extract(c, buf):
        @pl.loop(0, WCHUNK, step=L)
        def _(g):
            wv = widx_v[pl.ds(c * WCHUNK + g, L)]
            for j in range(L):
                sub = lax.bitwise_and(wv[j], 7)
                cidx = row_iota + sub * F
                ridx = jnp.full((L,), g + j, jnp.int32)
                wrow = plsc.load_gather(wbuf_v.at[buf], [ridx, cidx])
                wrows_v[pl.ds((c * WCHUNK + g + j) * F, F)] = wrow

    wcp = w_start(0, 0)
    for r in range(NRING):
        t_fire(r, r)

    for c in range(n_wchunks):
        nxt = w_start(c + 1, 1 - c % 2) if c + 1 < n_wchunks else None
        wcp.wait()
        w_extract(c, c % 2)
        wcp = nxt

    def t_group(g, ring):
        t_drain(ring)
        tv = tidx_v[pl.ds(g * L, L)]
        for j in range(L):
            col = lax.bitwise_and(tv[j], 127)
            cidx = jnp.full((L,), col, jnp.int32)
            tcol = plsc.load_gather(tring_v.at[ring, j], [row_iota, cidx])
            tg_v[pl.ds(j * F, F)] = tcol

        @pl.when(g + NRING < n_groups)
        def _():
            t_fire(g + NRING, ring)

        acc = jnp.zeros((L,), jnp.float32)
        for f in range(F):
            tcol = plsc.load_gather(tg_v, [lane16 + f])
            wcol = plsc.load_gather(wrows_v, [lane16 + (g * L * F + f)])
            acc = acc + tcol * wcol
        out_v[pl.ds(g * L, L)] = 1.0 / (1.0 + jnp.exp(-acc))

    @pl.loop(0, n_groups, step=NRING)
    def _(g):
        for r in range(NRING):
            @pl.when(g + r < n_groups)
            def _(r=r):
                t_group(g + r, r)

    pltpu.sync_copy(out_v, out_hbm.at[pl.ds(base, b_per_w)])


@jax.jit
def _mf_forward(task, worker, task_factors, worker_factors):
    B = task.shape[0]
    b_per_w = B // NW
    tfT = task_factors.T
    wfp = worker_factors.reshape(worker_factors.shape[0] // 8, 8 * F)
    mesh = plsc.VectorSubcoreMesh(core_axis_name="c", subcore_axis_name="s")
    cp = pltpu.CompilerParams(needs_layout_passes=False,
                              use_tc_tiling_on_sc=True)
    kern = functools.partial(
        pl.kernel,
        compiler_params=cp,
        out_type=jax.ShapeDtypeStruct((B,), jnp.float32),
        mesh=mesh,
        scratch_types=[
            pltpu.VMEM((b_per_w,), jnp.int32),
            pltpu.VMEM((b_per_w,), jnp.int32),
            pltpu.VMEM((b_per_w,), jnp.int32),
            pltpu.VMEM((NRING, L, F, 128), jnp.float32),
            pltpu.VMEM((2, WCHUNK, 8 * F), jnp.float32),
            pltpu.VMEM((L * F,), jnp.float32),
            pltpu.VMEM((b_per_w * F,), jnp.float32),
            pltpu.VMEM((b_per_w,), jnp.float32),
            pltpu.SemaphoreType.DMA((NRING + 2,)),
        ],
    )(_mf_kernel_body)
    return kern(task, worker, tfT, wfp)


def kernel(task, worker, task_factors, worker_factors):
    return _mf_forward(task, worker, task_factors, worker_factors)


# speedup vs baseline: 1.4914x; 1.0470x over previous
import functools

import jax
import jax.numpy as jnp
from jax import lax
from jax.experimental import pallas as pl
from jax.experimental.pallas import tpu as pltpu
from jax.experimental.pallas import tpu_sc as plsc

NC = 2
NS = 16
NW = NC * NS
L = 16
F = 16
WCHUNK = 64
NRING = 3


def _task_kernel_body(task_hbm, tfT_hbm, trows_hbm,
                      tidx_v, tring_v, trows_v, sems):
    b_per_w = tidx_v.shape[0]
    n_groups = b_per_w // L
    rows_per_w = b_per_w * F // 128
    wid = lax.axis_index("s") * NC + lax.axis_index("c")
    base = wid * b_per_w

    pltpu.sync_copy(task_hbm.at[pl.ds(base, b_per_w)], tidx_v)

    row_iota = lax.iota(jnp.int32, L)

    def t_fire(g, ring):
        tv = tidx_v[pl.ds(g * L, L)]
        for j in range(L):
            blk = pl.multiple_of(
                lax.shift_right_logical(tv[j], 7) * 128, 128)
            pltpu.async_copy(tfT_hbm.at[:, pl.ds(blk, 128)],
                             tring_v.at[ring, j], sems.at[ring])

    def t_drain(ring):
        for j in range(L):
            pltpu.make_async_copy(tfT_hbm.at[:, pl.ds(0, 128)],
                                  tring_v.at[ring, j], sems.at[ring]).wait()

    for r in range(NRING):
        t_fire(r, r)

    def t_group(g, ring):
        t_drain(ring)
        tv = tidx_v[pl.ds(g * L, L)]
        for j in range(L):
            col = lax.bitwise_and(tv[j], 127)
            cidx = jnp.full((L,), col, jnp.int32)
            tcol = plsc.load_gather(tring_v.at[ring, j], [row_iota, cidx])
            trows_v[j * F // 128, pl.ds((j * F) % 128, F)] = tcol

        @pl.when(g + NRING < n_groups)
        def _():
            t_fire(g + NRING, ring)

        pltpu.sync_copy(
            trows_v, trows_hbm.at[pl.ds(wid * rows_per_w + g * 2, 2)])

    @pl.loop(0, n_groups, step=NRING)
    def _(g):
        for r in range(NRING):
            @pl.when(g + r < n_groups)
            def _(r=r):
                t_group(g + r, r)


def _dot_kernel_body(worker_hbm, wfp_hbm, trows_hbm, out_hbm,
                     widx_v, wblk_v, wbuf_v, trows_v, wrows_v, out_v, sems):
    b_per_w = widx_v.shape[0]
    n_wchunks = b_per_w // WCHUNK
    rows_per_w = b_per_w * F // 128
    wid = lax.axis_index("s") * NC + lax.axis_index("c")
    base = wid * b_per_w

    pltpu.sync_copy(worker_hbm.at[pl.ds(base, b_per_w)], widx_v)
    pltpu.sync_copy(trows_hbm.at[pl.ds(wid * rows_per_w, rows_per_w)],
                    trows_v)

    row_iota = lax.iota(jnp.int32, L)

    @pl.loop(0, b_per_w, step=L)
    def _(g):
        wblk_v[pl.ds(g, L)] = lax.shift_right_logical(widx_v[pl.ds(g, L)], 3)

    def w_start(c, buf):
        sl = pl.ds(c * WCHUNK, WCHUNK)
        return pltpu.async_copy(wfp_hbm.at[wblk_v.at[sl]], wbuf_v.at[buf],
                                sems.at[buf])

    def w_extract(c, buf):
        @pl.loop(0, WCHUNK, step=L)
        def _(g):
            wv = widx_v[pl.ds(c * WCHUNK + g, L)]
            for j in range(L):
                sub = lax.bitwise_and(wv[j], 7)
                cidx = row_iota + sub * F
                ridx = jnp.full((L,), g + j, jnp.int32)
                wrow = plsc.load_gather(wbuf_v.at[buf], [ridx, cidx])
                wrows_v[pl.ds((c * WCHUNK + g + j) * F, F)] = wrow

    wcp = w_start(0, 0)
    for c in range(n_wchunks):
        nxt = w_start(c + 1, 1 - c % 2) if c + 1 < n_wchunks else None
        wcp.wait()
        w_extract(c, c % 2)
        wcp = nxt

    lane16 = row_iota * F

    @pl.loop(0, b_per_w, step=L)
    def _(p0):
        opos = row_iota + p0
        tr = lax.shift_right_logical(opos * F, 7)
        tc0 = lax.bitwise_and(opos * F, 127)
        acc = jnp.zeros((L,), jnp.float32)
        for f in range(F):
            tcol = plsc.load_gather(trows_v, [tr, tc0 + f])
            wcol = plsc.load_gather(wrows_v, [lane16 + (p0 * F + f)])
            acc = acc + tcol * wcol
        out_v[pl.ds(p0, L)] = 1.0 / (1.0 + jnp.exp(-acc))

    pltpu.sync_copy(out_v, out_hbm.at[pl.ds(base, b_per_w)])


@jax.jit
def _mf_forward(task, worker, task_factors, worker_factors):
    B = task.shape[0]
    b_per_w = B // NW
    rows_per_w = b_per_w * F // 128
    tfT = task_factors.T
    wfp = worker_factors.reshape(worker_factors.shape[0] // 8, 8 * F)
    mesh = plsc.VectorSubcoreMesh(core_axis_name="c", subcore_axis_name="s")
    cp = pltpu.CompilerParams(needs_layout_passes=False,
                              use_tc_tiling_on_sc=True)

    task_kern = functools.partial(
        pl.kernel,
        compiler_params=cp,
        out_type=jax.ShapeDtypeStruct((B * F // 128, 128), jnp.float32),
        mesh=mesh,
        scratch_types=[
            pltpu.VMEM((b_per_w,), jnp.int32),
            pltpu.VMEM((NRING, L, F, 128), jnp.float32),
            pltpu.VMEM((2, 128), jnp.float32),
            pltpu.SemaphoreType.DMA((NRING,)),
        ],
    )(_task_kernel_body)
    trows = task_kern(task, tfT)

    dot_kern = functools.partial(
        pl.kernel,
        compiler_params=cp,
        out_type=jax.ShapeDtypeStruct((B,), jnp.float32),
        mesh=mesh,
        scratch_types=[
            pltpu.VMEM((b_per_w,), jnp.int32),
            pltpu.VMEM((b_per_w,), jnp.int32),
            pltpu.VMEM((2, WCHUNK, 8 * F), jnp.float32),
            pltpu.VMEM((rows_per_w, 128), jnp.float32),
            pltpu.VMEM((b_per_w * F,), jnp.float32),
            pltpu.VMEM((b_per_w,), jnp.float32),
            pltpu.SemaphoreType.DMA((2,)),
        ],
    )(_dot_kernel_body)
    return dot_kern(worker, wfp, trows)


def kernel(task, worker, task_factors, worker_factors):
    return _mf_forward(task, worker, task_factors, worker_factors)
